# baseline (speedup 1.0000x reference)
import jax
import jax.numpy as jnp
from jax.experimental import pallas as pl

_RB = 6400


def _add_kernel(x_ref, pos_ref, o_ref):
    o_ref[...] = x_ref[...] + pos_ref[...]


def kernel(concat_embeddings, pos_table):
    b, s, h = concat_embeddings.shape
    rows = b * s
    x2 = concat_embeddings.reshape(rows, h)
    pos_tiled = jnp.tile(pos_table[:s], (_RB // s, 1))
    out = pl.pallas_call(
        _add_kernel,
        grid=(rows // _RB,),
        in_specs=[
            pl.BlockSpec((_RB, h), lambda i: (i, 0)),
            pl.BlockSpec((_RB, h), lambda i: (0, 0)),
        ],
        out_specs=pl.BlockSpec((_RB, h), lambda i: (i, 0)),
        out_shape=jax.ShapeDtypeStruct((rows, h), concat_embeddings.dtype),
    )(x2, pos_tiled)
    return out.reshape(b, s, h)


# speedup vs baseline: 2.1950x; 2.1950x over previous
import jax
import jax.numpy as jnp
from jax.experimental import pallas as pl
from jax.experimental.pallas import tpu as pltpu

_BB = 256


def _add_kernel(x_ref, pos_ref, o_ref, *, s):
    o_ref[...] = x_ref[...] + pos_ref[:s][None, :, :]


def kernel(concat_embeddings, pos_table):
    b, s, h = concat_embeddings.shape
    np_, _ = pos_table.shape
    import functools
    return pl.pallas_call(
        functools.partial(_add_kernel, s=s),
        grid=(b // _BB,),
        in_specs=[
            pl.BlockSpec((_BB, s, h), lambda i: (i, 0, 0)),
            pl.BlockSpec((np_, h), lambda i: (0, 0)),
        ],
        out_specs=pl.BlockSpec((_BB, s, h), lambda i: (i, 0, 0)),
        out_shape=jax.ShapeDtypeStruct((b, s, h), concat_embeddings.dtype),
        compiler_params=pltpu.CompilerParams(
            dimension_semantics=("arbitrary",),
        ),
    )(concat_embeddings, pos_table)


# speedup vs baseline: 2.1973x; 1.0010x over previous
import jax
import jax.numpy as jnp
from jax.experimental import pallas as pl
from jax.experimental.pallas import tpu as pltpu

_BB = 256


def _add_kernel(x_ref, pos_ref, o_ref, *, s):
    o_ref[...] = x_ref[...] + pos_ref[:s][None, :, :]


def kernel(concat_embeddings, pos_table):
    b, s, h = concat_embeddings.shape
    np_, _ = pos_table.shape
    import functools
    return pl.pallas_call(
        functools.partial(_add_kernel, s=s),
        grid=(b // _BB,),
        in_specs=[
            pl.BlockSpec((_BB, s, h), lambda i: (i, 0, 0)),
            pl.BlockSpec((np_, h), lambda i: (0, 0)),
        ],
        out_specs=pl.BlockSpec((_BB, s, h), lambda i: (i, 0, 0)),
        out_shape=jax.ShapeDtypeStruct((b, s, h), concat_embeddings.dtype),
        compiler_params=pltpu.CompilerParams(
            dimension_semantics=("parallel",),
        ),
    )(concat_embeddings, pos_table)


# speedup vs baseline: 2.2827x; 1.0389x over previous
import jax
import jax.numpy as jnp
from jax.experimental import pallas as pl
from jax.experimental.pallas import tpu as pltpu

_CB = 128
_NBUF = 4


def _add_pos_kernel(x_hbm, pos_hbm, out_hbm, x_vmem, o_vmem, pos_vmem,
                    in_sems, out_sems, pos_sem):
    nb = x_hbm.shape[0]
    nc = nb // _CB
    s = x_hbm.shape[1]

    pltpu.make_async_copy(pos_hbm, pos_vmem, pos_sem).start()

    def in_copy(i, slot):
        return pltpu.make_async_copy(
            x_hbm.at[pl.ds(i * _CB, _CB)], x_vmem.at[slot], in_sems.at[slot])

    def out_copy(i, slot):
        return pltpu.make_async_copy(
            o_vmem.at[slot], out_hbm.at[pl.ds(i * _CB, _CB)], out_sems.at[slot])

    for k in range(min(_NBUF, nc)):
        in_copy(k, k).start(priority=0)

    pltpu.make_async_copy(pos_hbm, pos_vmem, pos_sem).wait()
    pos = pos_vmem[:s, :][None, :, :]

    for i in range(nc):
        slot = i % _NBUF
        in_copy(i, slot).wait()
        if i >= _NBUF:
            out_copy(i - _NBUF, slot).wait()
        o_vmem[slot] = x_vmem[slot] + pos
        out_copy(i, slot).start(priority=1)
        if i + _NBUF < nc:
            in_copy(i + _NBUF, slot).start(priority=0)

    for i in range(max(nc - _NBUF, 0), nc):
        out_copy(i, i % _NBUF).wait()


def kernel(concat_embeddings, pos_table):
    b, s, h = concat_embeddings.shape
    np_, _ = pos_table.shape
    return pl.pallas_call(
        _add_pos_kernel,
        in_specs=[
            pl.BlockSpec(memory_space=pltpu.MemorySpace.HBM),
            pl.BlockSpec(memory_space=pltpu.MemorySpace.HBM),
        ],
        out_specs=pl.BlockSpec(memory_space=pltpu.MemorySpace.HBM),
        out_shape=jax.ShapeDtypeStruct((b, s, h), concat_embeddings.dtype),
        scratch_shapes=[
            pltpu.VMEM((_NBUF, _CB, s, h), concat_embeddings.dtype),
            pltpu.VMEM((_NBUF, _CB, s, h), concat_embeddings.dtype),
            pltpu.VMEM((np_, h), pos_table.dtype),
            pltpu.SemaphoreType.DMA((_NBUF,)),
            pltpu.SemaphoreType.DMA((_NBUF,)),
            pltpu.SemaphoreType.DMA,
        ],
    )(concat_embeddings, pos_table)
